# baseline (speedup 1.0000x reference)
import jax
import jax.numpy as jnp
from jax import lax
from jax.experimental import pallas as pl
from jax.experimental.pallas import tpu as pltpu
from jax.experimental.pallas import tpu_sc as plsc

N = 10000
E = 160000
D = 256
H = 128
NR = 200

NC = 2
NS = 16
NW = NC * NS

ROWB = 1000

CA = 96
NKA = 53
EPADA = NW * NKA * CA

CB = 80
NKB = 125
NKBP = 128
GB = 2
NGB = NKBP // GB


def _dense_body(x_ref, w_ref, p1_ref, p2_ref, m1a_ref, m1b_ref):
    z = jnp.dot(x_ref[...], w_ref[...], preferred_element_type=jnp.float32)
    p1_ref[...] = z[:, :D]
    p2_ref[...] = z[:, D:2 * D]
    m1a_ref[...] = z[:, 2 * D:2 * D + H]
    m1b_ref[...] = z[:, 2 * D + H:]


def _dense_tables(prev_h, w_all):
    return pl.pallas_call(
        _dense_body,
        grid=(N // ROWB,),
        in_specs=[
            pl.BlockSpec((ROWB, D), lambda i: (i, 0)),
            pl.BlockSpec((D, 3 * D), lambda i: (0, 0)),
        ],
        out_specs=[
            pl.BlockSpec((ROWB, D), lambda i: (i, 0)),
            pl.BlockSpec((ROWB, D), lambda i: (i, 0)),
            pl.BlockSpec((ROWB, H), lambda i: (i, 0)),
            pl.BlockSpec((ROWB, H), lambda i: (i, 0)),
        ],
        out_shape=[
            jax.ShapeDtypeStruct((N, D), jnp.float32),
            jax.ShapeDtypeStruct((N, D), jnp.float32),
            jax.ShapeDtypeStruct((N, H), jnp.float32),
            jax.ShapeDtypeStruct((N, H), jnp.float32),
        ],
    )(prev_h, w_all)


def _rel_body(x_ref, w_ref, ra_ref, rb_ref):
    z = jnp.dot(x_ref[...], w_ref[...], preferred_element_type=jnp.float32)
    ra_ref[...] = z[:, :H]
    rb_ref[...] = z[:, H:]


def _rel_tables(emb_rel, wn2):
    return pl.pallas_call(
        _rel_body,
        out_shape=[
            jax.ShapeDtypeStruct((NR, H), jnp.float32),
            jax.ShapeDtypeStruct((NR, H), jnp.float32),
        ],
    )(emb_rel, wn2)


def _attn_body(pa_hbm, p1_hbm, p2_hbm, wv_hbm, ex_hbm,
               pab, p1b0, p2b0, p1b1, p2b1, eb, wv, hbuf, sem0, sem1):
    c = lax.axis_index("c")
    s = lax.axis_index("s")
    wid = s * NC + c
    pltpu.sync_copy(wv_hbm, wv)
    pltpu.sync_copy(pa_hbm.at[pl.ds(wid * (2 * NKA * CA), 2 * NKA * CA)], pab)
    bufs = ((p1b0, p2b0, sem0), (p1b1, p2b1, sem1))
    lanes = lax.iota(jnp.int32, 16)

    def issue(b, k):
        p1b, p2b, sem = bufs[b]

        @pl.when(k < NKA)
        def _():
            pltpu.async_copy(
                p1_hbm.at[pab.at[pl.ds(2 * CA * k, CA)]], p1b, sem)
            pltpu.async_copy(
                p2_hbm.at[pab.at[pl.ds(2 * CA * k + CA, CA)]], p2b, sem)

    def compute(b, k):
        p1b, p2b, sem = bufs[b]

        @pl.when(k < NKA)
        def _():
            pltpu.make_async_copy(
                p1_hbm.at[pab.at[pl.ds(0, CA)]], p1b, sem).wait()
            pltpu.make_async_copy(
                p2_hbm.at[pab.at[pl.ds(0, CA)]], p2b, sem).wait()
            for g in range(CA // 16):
                def edge_body(i, v):
                    r = g * 16 + i
                    acc = jnp.zeros((16,), jnp.float32)
                    for j in range(D // 16):
                        a = (p1b[r, pl.ds(16 * j, 16)]
                             + p2b[r, pl.ds(16 * j, 16)])
                        a = jnp.maximum(a, 0.01 * a)
                        acc = acc + a * wv[pl.ds(16 * j, 16)]
                    for sh in (8, 4, 2, 1):
                        hbuf[pl.ds(0, 16)] = acc
                        hbuf[pl.ds(16, 16)] = acc
                        acc = acc + hbuf[pl.ds(sh, 16)]
                    return jnp.where(lanes == i, acc, v)

                v = lax.fori_loop(0, 16, edge_body,
                                  jnp.zeros((16,), jnp.float32))
                eb[pl.ds(16 * g, 16)] = jnp.exp(v)
            pltpu.sync_copy(eb, ex_hbm.at[pl.ds((wid * NKA + k) * CA, CA)])

    issue(0, 0)

    def pair(p, carry):
        k0 = 2 * p
        issue(1, k0 + 1)
        compute(0, k0)
        issue(0, k0 + 2)
        compute(1, k0 + 1)
        return carry

    lax.fori_loop(0, (NKA + 1) // 2, pair, 0)


def _attn_pass(pa, p1, p2, wv):
    mesh = plsc.VectorSubcoreMesh(core_axis_name="c", subcore_axis_name="s")
    f = pl.kernel(
        _attn_body,
        out_type=jax.ShapeDtypeStruct((EPADA,), jnp.float32),
        mesh=mesh,
        scratch_types=[
            pltpu.VMEM((2 * NKA * CA,), jnp.int32),
            pltpu.VMEM((CA, D), jnp.float32),
            pltpu.VMEM((CA, D), jnp.float32),
            pltpu.VMEM((CA, D), jnp.float32),
            pltpu.VMEM((CA, D), jnp.float32),
            pltpu.VMEM((CA,), jnp.float32),
            pltpu.VMEM((D,), jnp.float32),
            pltpu.VMEM((32,), jnp.float32),
            pltpu.SemaphoreType.DMA,
            pltpu.SemaphoreType.DMA,
        ],
    )
    return f(pa, p1, p2, wv)


ZR = 25
RPS = N // NS
DPAD = 10240


def _agg_body(pb_hbm, pe_hbm, m1a_hbm, m1b_hbm, r2a_hbm, r2b_hbm,
              outa_hbm, outb_hbm,
              gb0, ge0, gb1, ge1, m1v0, m1v1, r2v, dstw,
              zb, db, dvb, acc_sh, den_sh,
              isem0, isem1, sem0, sem1):
    c = lax.axis_index("c")
    s = lax.axis_index("s")

    @pl.when(c == 0)
    def _():
        pltpu.sync_copy(r2a_hbm, r2v)

    @pl.when(c == 1)
    def _():
        pltpu.sync_copy(r2b_hbm, r2v)

    def zrow_body(i, carry):
        for j in range(H // 16):
            zb[i, pl.ds(16 * j, 16)] = jnp.zeros((16,), jnp.float32)
        return carry

    lax.fori_loop(0, 16, zrow_body, 0)

    def dz_body(i, carry):
        db[pl.ds(16 * i, 16)] = jnp.zeros((16,), jnp.float32)
        return carry

    lax.fori_loop(0, 8, dz_body, 0)

    def zcp_body(t, carry):
        bid = s + t * NS

        @pl.when(bid < N // 16)
        def _():
            pltpu.sync_copy(zb, acc_sh.at[pl.ds(bid * 16, 16)])
        return carry

    lax.fori_loop(0, (N // 16 + NS - 1) // NS, zcp_body, 0)

    def dcp_body(kk, carry):
        pltpu.sync_copy(db, den_sh.at[pl.ds(s * (DPAD // NS) + kk * 128, 128)])
        return carry

    lax.fori_loop(0, (DPAD // NS) // 128, dcp_body, 0)
    plsc.subcore_barrier()

    gbufs = ((gb0, ge0, isem0), (gb1, ge1, isem1))
    cbufs = ((m1v0, sem0), (m1v1, sem1))

    def fetchg(gbi, gi):
        gb, ge, isem = gbufs[gbi]

        @pl.when(gi < NGB)
        def _():
            pltpu.async_copy(
                pb_hbm.at[pl.ds((s * NKBP + GB * gi) * 3 * CB, 3 * GB * CB)],
                gb, isem)
            pltpu.async_copy(
                pe_hbm.at[pl.ds((s * NKBP + GB * gi) * CB, GB * CB)],
                ge, isem)

    def waitg(gbi):
        gb, ge, isem = gbufs[gbi]
        pltpu.make_async_copy(
            pb_hbm.at[pl.ds(0, 3 * GB * CB)], gb, isem).wait()
        pltpu.make_async_copy(
            pe_hbm.at[pl.ds(0, GB * CB)], ge, isem).wait()

    def issue(cbi, gbi, jj, k):
        m1v, sem = cbufs[cbi]
        gb, _, _ = gbufs[gbi]

        @pl.when(k < NKB)
        def _():
            srcix = gb.at[pl.ds(3 * CB * jj, CB)]

            @pl.when(c == 0)
            def _():
                pltpu.async_copy(m1a_hbm.at[srcix], m1v, sem)

            @pl.when(c == 1)
            def _():
                pltpu.async_copy(m1b_hbm.at[srcix], m1v, sem)

    def compute(cbi, gbi, jj, k):
        m1v, sem = cbufs[cbi]
        gb, ge, _ = gbufs[gbi]

        @pl.when(k < NKB)
        def _():
            dummy = gb.at[pl.ds(0, CB)]
            pltpu.make_async_copy(m1a_hbm.at[dummy], m1v, sem).wait()
            def grp_body(g2, carry2):
                off = 3 * CB * jj + 16 * g2
                exv = ge[pl.ds(CB * jj + 16 * g2, 16)]
                rv = gb[pl.ds(off + 2 * CB, 16)]
                dstw[pl.ds(16 * g2, 16)] = gb[pl.ds(off + CB, 16)]
                base = 16 * g2
                for i in range(16):
                    exi = jnp.full((16,), exv[i], jnp.float32)
                    ri = rv[i]
                    r = base + i
                    for j in range(H // 16):
                        rel = r2v[ri, pl.ds(16 * j, 16)]
                        m1v[r, pl.ds(16 * j, 16)] = (
                            m1v[r, pl.ds(16 * j, 16)] + rel) * exi
                return carry2

            lax.fori_loop(0, CB // 16, grp_body, 0)
            pltpu.sync_copy(m1v, acc_sh.at[dstw], add=True)
            pltpu.sync_copy(ge.at[pl.ds(CB * jj, CB)], den_sh.at[dstw],
                            add=True)

    fetchg(0, 0)
    fetchg(1, 1)

    def pairg(p, carry):
        g0 = 2 * p
        g1 = 2 * p + 1
        waitg(0)
        issue(0, 0, 0, g0 * GB + 0)
        issue(1, 0, 1, g0 * GB + 1)
        compute(0, 0, 0, g0 * GB + 0)
        waitg(1)
        issue(0, 1, 0, g1 * GB + 0)
        compute(1, 0, 1, g0 * GB + 1)
        fetchg(0, g0 + 2)
        issue(1, 1, 1, g1 * GB + 1)
        compute(0, 1, 0, g1 * GB + 0)
        compute(1, 1, 1, g1 * GB + 1)
        fetchg(1, g1 + 2)
        return carry

    lax.fori_loop(0, NGB // 2, pairg, 0)
    plsc.subcore_barrier()

    nblk = N // 16

    def wblk_body(t, carry):
        bid = s + t * NS

        @pl.when(bid < nblk)
        def _():
            r0 = bid * 16
            pltpu.sync_copy(acc_sh.at[pl.ds(r0, 16)], zb)
            pltpu.sync_copy(den_sh.at[pl.ds(r0, 16)], dvb)
            dv = dvb[pl.ds(0, 16)]
            rv16 = jnp.where(dv == 0.0, 0.0, 1.0 / dv)
            for i in range(16):
                rv = jnp.full((16,), rv16[i], jnp.float32)
                for j in range(H // 16):
                    zb[i, pl.ds(16 * j, 16)] = zb[i, pl.ds(16 * j, 16)] * rv

            @pl.when(c == 0)
            def _():
                pltpu.sync_copy(zb, outa_hbm.at[pl.ds(r0, 16)])

            @pl.when(c == 1)
            def _():
                pltpu.sync_copy(zb, outb_hbm.at[pl.ds(r0, 16)])

        return carry

    lax.fori_loop(0, (nblk + NS - 1) // NS, wblk_body, 0)


def _agg_pass(pb, pe, m1a, m1b, r2a, r2b):
    mesh = plsc.VectorSubcoreMesh(core_axis_name="c", subcore_axis_name="s")
    f = pl.kernel(
        _agg_body,
        out_type=[
            jax.ShapeDtypeStruct((N, H), jnp.float32),
            jax.ShapeDtypeStruct((N, H), jnp.float32),
        ],
        mesh=mesh,
        scratch_types=[
            pltpu.VMEM((3 * GB * CB,), jnp.int32),
            pltpu.VMEM((GB * CB,), jnp.float32),
            pltpu.VMEM((3 * GB * CB,), jnp.int32),
            pltpu.VMEM((GB * CB,), jnp.float32),
            pltpu.VMEM((CB, H), jnp.float32),
            pltpu.VMEM((CB, H), jnp.float32),
            pltpu.VMEM((NR, H), jnp.float32),
            pltpu.VMEM((CB,), jnp.int32),
            pltpu.VMEM((16, H), jnp.float32),
            pltpu.VMEM((128,), jnp.float32),
            pltpu.VMEM((16,), jnp.float32),
            pltpu.VMEM_SHARED((N, H), jnp.float32),
            pltpu.VMEM_SHARED((DPAD,), jnp.float32),
            pltpu.SemaphoreType.DMA,
            pltpu.SemaphoreType.DMA,
            pltpu.SemaphoreType.DMA,
            pltpu.SemaphoreType.DMA,
        ],
    )
    return f(pb, pe, m1a, m1b, r2a, r2b)


@jax.jit
def _run(prev_h, emb_rel, edge_index, rid, pos_proj_w, attn_fc_w,
         weight_neighbor):
    w_all = jnp.concatenate(
        [pos_proj_w[:D], pos_proj_w[D:], weight_neighbor[:D]], axis=1)
    p1, p2, m1a, m1b = _dense_tables(prev_h, w_all)
    r2a, r2b = _rel_tables(emb_rel, weight_neighbor[D:])
    src = edge_index[0]
    dst = edge_index[1]

    def pad_chunks_a(x):
        xw = x.reshape(NW, E // NW)
        xw = jnp.pad(xw, ((0, 0), (0, NKA * CA - E // NW)))
        return xw.reshape(NW, NKA, 1, CA)

    pa = jnp.concatenate([pad_chunks_a(src), pad_chunks_a(dst)], axis=2)
    pa = pa.reshape(NW * NKA * 2 * CA)
    ex = _attn_pass(pa, p1, p2, attn_fc_w[:, 0])
    ex = ex.reshape(NW, NKA * CA)[:, :E // NW].reshape(E)

    def pad_chunks_b(x):
        xw = x.reshape(NS, NKB, 1, CB)
        return jnp.pad(xw, ((0, 0), (0, NKBP - NKB), (0, 0), (0, 0)))

    pb = jnp.concatenate(
        [pad_chunks_b(src), pad_chunks_b(dst), pad_chunks_b(rid)], axis=2)
    pb = pb.reshape(NS * NKBP * 3 * CB)
    pe = pad_chunks_b(ex).reshape(NS * NKBP * CB)
    outa, outb = _agg_pass(pb, pe, m1a, m1b, r2a, r2b)
    return jnp.concatenate([outa, outb], axis=1)


def kernel(prev_h, emb_rel, edge_index, rid, pos_proj_w, attn_fc_w,
           weight_neighbor, k):
    return _run(prev_h, emb_rel, edge_index, rid, pos_proj_w, attn_fc_w,
                weight_neighbor)


# speedup vs baseline: 1.4096x; 1.4096x over previous
import jax
import jax.numpy as jnp
from jax import lax
from jax.experimental import pallas as pl
from jax.experimental.pallas import tpu as pltpu
from jax.experimental.pallas import tpu_sc as plsc

N = 10000
E = 160000
D = 256
H = 128
NR = 200

NC = 2
NS = 16
NW = NC * NS

ROWB = 1000

CA = 96
NKA = 53
EPADA = NW * NKA * CA

CB = 80
NKB = 125
NKBP = 128
GB = 4
NGB = NKBP // GB


def _dense_body(x_ref, w_ref, p1_ref, p2_ref, m1a_ref, m1b_ref):
    z = jnp.dot(x_ref[...], w_ref[...], preferred_element_type=jnp.float32)
    p1_ref[...] = z[:, :D]
    p2_ref[...] = z[:, D:2 * D]
    m1a_ref[...] = z[:, 2 * D:2 * D + H]
    m1b_ref[...] = z[:, 2 * D + H:]


def _dense_tables(prev_h, w_all):
    return pl.pallas_call(
        _dense_body,
        grid=(N // ROWB,),
        in_specs=[
            pl.BlockSpec((ROWB, D), lambda i: (i, 0)),
            pl.BlockSpec((D, 3 * D), lambda i: (0, 0)),
        ],
        out_specs=[
            pl.BlockSpec((ROWB, D), lambda i: (i, 0)),
            pl.BlockSpec((ROWB, D), lambda i: (i, 0)),
            pl.BlockSpec((ROWB, H), lambda i: (i, 0)),
            pl.BlockSpec((ROWB, H), lambda i: (i, 0)),
        ],
        out_shape=[
            jax.ShapeDtypeStruct((N, D), jnp.float32),
            jax.ShapeDtypeStruct((N, D), jnp.float32),
            jax.ShapeDtypeStruct((N, H), jnp.float32),
            jax.ShapeDtypeStruct((N, H), jnp.float32),
        ],
    )(prev_h, w_all)


def _rel_body(x_ref, w_ref, ra_ref, rb_ref):
    z = jnp.dot(x_ref[...], w_ref[...], preferred_element_type=jnp.float32)
    ra_ref[...] = z[:, :H]
    rb_ref[...] = z[:, H:]


def _rel_tables(emb_rel, wn2):
    return pl.pallas_call(
        _rel_body,
        out_shape=[
            jax.ShapeDtypeStruct((NR, H), jnp.float32),
            jax.ShapeDtypeStruct((NR, H), jnp.float32),
        ],
    )(emb_rel, wn2)


def _attn_body(pa_hbm, p1_hbm, p2_hbm, wv_hbm, ex_hbm,
               pab, p1b0, p2b0, p1b1, p2b1, eb, wv, hbuf, sem0, sem1):
    c = lax.axis_index("c")
    s = lax.axis_index("s")
    wid = s * NC + c
    pltpu.sync_copy(wv_hbm, wv)
    pltpu.sync_copy(pa_hbm.at[pl.ds(wid * (2 * NKA * CA), 2 * NKA * CA)], pab)
    bufs = ((p1b0, p2b0, sem0), (p1b1, p2b1, sem1))
    lanes = lax.iota(jnp.int32, 16)

    def issue(b, k):
        p1b, p2b, sem = bufs[b]

        @pl.when(k < NKA)
        def _():
            pltpu.async_copy(
                p1_hbm.at[pab.at[pl.ds(2 * CA * k, CA)]], p1b, sem)
            pltpu.async_copy(
                p2_hbm.at[pab.at[pl.ds(2 * CA * k + CA, CA)]], p2b, sem)

    def compute(b, k):
        p1b, p2b, sem = bufs[b]

        @pl.when(k < NKA)
        def _():
            pltpu.make_async_copy(
                p1_hbm.at[pab.at[pl.ds(0, CA)]], p1b, sem).wait()
            pltpu.make_async_copy(
                p2_hbm.at[pab.at[pl.ds(0, CA)]], p2b, sem).wait()
            for g in range(CA // 16):
                def edge_body(i, v):
                    r = g * 16 + i
                    acc = jnp.zeros((16,), jnp.float32)
                    for j in range(D // 16):
                        a = (p1b[r, pl.ds(16 * j, 16)]
                             + p2b[r, pl.ds(16 * j, 16)])
                        a = jnp.maximum(a, 0.01 * a)
                        acc = acc + a * wv[pl.ds(16 * j, 16)]
                    for sh in (8, 4, 2, 1):
                        hbuf[pl.ds(0, 16)] = acc
                        hbuf[pl.ds(16, 16)] = acc
                        acc = acc + hbuf[pl.ds(sh, 16)]
                    return jnp.where(lanes == i, acc, v)

                v = lax.fori_loop(0, 16, edge_body,
                                  jnp.zeros((16,), jnp.float32))
                eb[pl.ds(16 * g, 16)] = jnp.exp(v)
            pltpu.sync_copy(eb, ex_hbm.at[pl.ds((wid * NKA + k) * CA, CA)])

    issue(0, 0)

    def pair(p, carry):
        k0 = 2 * p
        issue(1, k0 + 1)
        compute(0, k0)
        issue(0, k0 + 2)
        compute(1, k0 + 1)
        return carry

    lax.fori_loop(0, (NKA + 1) // 2, pair, 0)


def _attn_pass(pa, p1, p2, wv):
    mesh = plsc.VectorSubcoreMesh(core_axis_name="c", subcore_axis_name="s")
    f = pl.kernel(
        _attn_body,
        out_type=jax.ShapeDtypeStruct((EPADA,), jnp.float32),
        mesh=mesh,
        scratch_types=[
            pltpu.VMEM((2 * NKA * CA,), jnp.int32),
            pltpu.VMEM((CA, D), jnp.float32),
            pltpu.VMEM((CA, D), jnp.float32),
            pltpu.VMEM((CA, D), jnp.float32),
            pltpu.VMEM((CA, D), jnp.float32),
            pltpu.VMEM((CA,), jnp.float32),
            pltpu.VMEM((D,), jnp.float32),
            pltpu.VMEM((32,), jnp.float32),
            pltpu.SemaphoreType.DMA,
            pltpu.SemaphoreType.DMA,
        ],
    )
    return f(pa, p1, p2, wv)


ZR = 25
RPS = N // NS
DPAD = 10240


def _agg_body(pb_hbm, pe_hbm, m1a_hbm, m1b_hbm, r2a_hbm, r2b_hbm,
              outa_hbm, outb_hbm,
              gb0, ge0, gb1, ge1, m1v0, relb0, m1v1, relb1, dstw,
              zb, db, dvb, acc_sh, den_sh,
              isem0, isem1, sem0, sem1):
    c = lax.axis_index("c")
    s = lax.axis_index("s")

    def zrow_body(i, carry):
        for j in range(H // 16):
            zb[i, pl.ds(16 * j, 16)] = jnp.zeros((16,), jnp.float32)
        return carry

    lax.fori_loop(0, 16, zrow_body, 0)

    def dz_body(i, carry):
        db[pl.ds(16 * i, 16)] = jnp.zeros((16,), jnp.float32)
        return carry

    lax.fori_loop(0, 8, dz_body, 0)

    def zcp_body(t, carry):
        bid = s + t * NS

        @pl.when(bid < N // 16)
        def _():
            pltpu.sync_copy(zb, acc_sh.at[pl.ds(bid * 16, 16)])
        return carry

    lax.fori_loop(0, (N // 16 + NS - 1) // NS, zcp_body, 0)

    def dcp_body(kk, carry):
        pltpu.sync_copy(db, den_sh.at[pl.ds(s * (DPAD // NS) + kk * 128, 128)])
        return carry

    lax.fori_loop(0, (DPAD // NS) // 128, dcp_body, 0)
    plsc.subcore_barrier()

    gbufs = ((gb0, ge0, isem0), (gb1, ge1, isem1))
    cbufs = ((m1v0, relb0, sem0), (m1v1, relb1, sem1))

    def fetchg(gbi, gi):
        gb, ge, isem = gbufs[gbi]

        @pl.when(gi < NGB)
        def _():
            pltpu.async_copy(
                pb_hbm.at[pl.ds((s * NKBP + GB * gi) * 3 * CB, 3 * GB * CB)],
                gb, isem)
            pltpu.async_copy(
                pe_hbm.at[pl.ds((s * NKBP + GB * gi) * CB, GB * CB)],
                ge, isem)

    def waitg(gbi):
        gb, ge, isem = gbufs[gbi]
        pltpu.make_async_copy(
            pb_hbm.at[pl.ds(0, 3 * GB * CB)], gb, isem).wait()
        pltpu.make_async_copy(
            pe_hbm.at[pl.ds(0, GB * CB)], ge, isem).wait()

    def issue(cbi, gbi, jj, k):
        m1v, relb, sem = cbufs[cbi]
        gb, _, _ = gbufs[gbi]

        @pl.when(k < NKB)
        def _():
            srcix = gb.at[pl.ds(3 * CB * jj, CB)]
            ridix = gb.at[pl.ds(3 * CB * jj + 2 * CB, CB)]

            @pl.when(c == 0)
            def _():
                pltpu.async_copy(m1a_hbm.at[srcix], m1v, sem)
                pltpu.async_copy(r2a_hbm.at[ridix], relb, sem)

            @pl.when(c == 1)
            def _():
                pltpu.async_copy(m1b_hbm.at[srcix], m1v, sem)
                pltpu.async_copy(r2b_hbm.at[ridix], relb, sem)

    def compute(cbi, gbi, jj, k):
        m1v, relb, sem = cbufs[cbi]
        gb, ge, _ = gbufs[gbi]

        @pl.when(k < NKB)
        def _():
            dummy = gb.at[pl.ds(0, CB)]
            pltpu.make_async_copy(m1a_hbm.at[dummy], m1v, sem).wait()
            pltpu.make_async_copy(r2a_hbm.at[dummy], relb, sem).wait()

            def grp_body(g2, carry2):
                off = 3 * CB * jj + 16 * g2
                exv = ge[pl.ds(CB * jj + 16 * g2, 16)]
                dstw[pl.ds(16 * g2, 16)] = gb[pl.ds(off + CB, 16)]
                base = 16 * g2
                for i in range(16):
                    exi = jnp.full((16,), exv[i], jnp.float32)
                    r = base + i
                    for j in range(H // 16):
                        m1v[r, pl.ds(16 * j, 16)] = (
                            m1v[r, pl.ds(16 * j, 16)]
                            + relb[r, pl.ds(16 * j, 16)]
                        ) * exi
                return carry2

            lax.fori_loop(0, CB // 16, grp_body, 0)
            pltpu.sync_copy(m1v, acc_sh.at[dstw], add=True)
            pltpu.sync_copy(ge.at[pl.ds(CB * jj, CB)], den_sh.at[dstw],
                            add=True)

    fetchg(0, 0)
    fetchg(1, 1)

    def pairg(p, carry):
        g0 = 2 * p
        g1 = 2 * p + 1
        waitg(0)
        issue(0, 0, 0, g0 * GB + 0)
        issue(1, 0, 1, g0 * GB + 1)
        compute(0, 0, 0, g0 * GB + 0)
        issue(0, 0, 2, g0 * GB + 2)
        compute(1, 0, 1, g0 * GB + 1)
        issue(1, 0, 3, g0 * GB + 3)
        compute(0, 0, 2, g0 * GB + 2)
        waitg(1)
        issue(0, 1, 0, g1 * GB + 0)
        compute(1, 0, 3, g0 * GB + 3)
        fetchg(0, g0 + 2)
        issue(1, 1, 1, g1 * GB + 1)
        compute(0, 1, 0, g1 * GB + 0)
        issue(0, 1, 2, g1 * GB + 2)
        compute(1, 1, 1, g1 * GB + 1)
        issue(1, 1, 3, g1 * GB + 3)
        compute(0, 1, 2, g1 * GB + 2)
        compute(1, 1, 3, g1 * GB + 3)
        fetchg(1, g1 + 2)
        return carry

    lax.fori_loop(0, NGB // 2, pairg, 0)
    plsc.subcore_barrier()

    nblk = N // 16

    def wblk_body(t, carry):
        bid = s + t * NS

        @pl.when(bid < nblk)
        def _():
            r0 = bid * 16
            pltpu.sync_copy(acc_sh.at[pl.ds(r0, 16)], zb)
            pltpu.sync_copy(den_sh.at[pl.ds(r0, 16)], dvb)
            dv = dvb[pl.ds(0, 16)]
            rv16 = jnp.where(dv == 0.0, 0.0, 1.0 / dv)
            for i in range(16):
                rv = jnp.full((16,), rv16[i], jnp.float32)
                for j in range(H // 16):
                    zb[i, pl.ds(16 * j, 16)] = zb[i, pl.ds(16 * j, 16)] * rv

            @pl.when(c == 0)
            def _():
                pltpu.sync_copy(zb, outa_hbm.at[pl.ds(r0, 16)])

            @pl.when(c == 1)
            def _():
                pltpu.sync_copy(zb, outb_hbm.at[pl.ds(r0, 16)])

        return carry

    lax.fori_loop(0, (nblk + NS - 1) // NS, wblk_body, 0)


def _agg_pass(pb, pe, m1a, m1b, r2a, r2b):
    mesh = plsc.VectorSubcoreMesh(core_axis_name="c", subcore_axis_name="s")
    f = pl.kernel(
        _agg_body,
        out_type=[
            jax.ShapeDtypeStruct((N, H), jnp.float32),
            jax.ShapeDtypeStruct((N, H), jnp.float32),
        ],
        mesh=mesh,
        scratch_types=[
            pltpu.VMEM((3 * GB * CB,), jnp.int32),
            pltpu.VMEM((GB * CB,), jnp.float32),
            pltpu.VMEM((3 * GB * CB,), jnp.int32),
            pltpu.VMEM((GB * CB,), jnp.float32),
            pltpu.VMEM((CB, H), jnp.float32),
            pltpu.VMEM((CB, H), jnp.float32),
            pltpu.VMEM((CB, H), jnp.float32),
            pltpu.VMEM((CB, H), jnp.float32),
            pltpu.VMEM((CB,), jnp.int32),
            pltpu.VMEM((16, H), jnp.float32),
            pltpu.VMEM((128,), jnp.float32),
            pltpu.VMEM((16,), jnp.float32),
            pltpu.VMEM_SHARED((N, H), jnp.float32),
            pltpu.VMEM_SHARED((DPAD,), jnp.float32),
            pltpu.SemaphoreType.DMA,
            pltpu.SemaphoreType.DMA,
            pltpu.SemaphoreType.DMA,
            pltpu.SemaphoreType.DMA,
        ],
    )
    return f(pb, pe, m1a, m1b, r2a, r2b)


@jax.jit
def _run(prev_h, emb_rel, edge_index, rid, pos_proj_w, attn_fc_w,
         weight_neighbor):
    w_all = jnp.concatenate(
        [pos_proj_w[:D], pos_proj_w[D:], weight_neighbor[:D]], axis=1)
    p1, p2, m1a, m1b = _dense_tables(prev_h, w_all)
    r2a, r2b = _rel_tables(emb_rel, weight_neighbor[D:])
    src = edge_index[0]
    dst = edge_index[1]

    def pad_chunks_a(x):
        xw = x.reshape(NW, E // NW)
        xw = jnp.pad(xw, ((0, 0), (0, NKA * CA - E // NW)))
        return xw.reshape(NW, NKA, 1, CA)

    pa = jnp.concatenate([pad_chunks_a(src), pad_chunks_a(dst)], axis=2)
    pa = pa.reshape(NW * NKA * 2 * CA)
    ex = _attn_pass(pa, p1, p2, attn_fc_w[:, 0])
    ex = ex.reshape(NW, NKA * CA)[:, :E // NW].reshape(E)

    def pad_chunks_b(x):
        xw = x.reshape(NS, NKB, 1, CB)
        return jnp.pad(xw, ((0, 0), (0, NKBP - NKB), (0, 0), (0, 0)))

    pb = jnp.concatenate(
        [pad_chunks_b(src), pad_chunks_b(dst), pad_chunks_b(rid)], axis=2)
    pb = pb.reshape(NS * NKBP * 3 * CB)
    pe = pad_chunks_b(ex).reshape(NS * NKBP * CB)
    outa, outb = _agg_pass(pb, pe, m1a, m1b, r2a, r2b)
    return jnp.concatenate([outa, outb], axis=1)


def kernel(prev_h, emb_rel, edge_index, rid, pos_proj_w, attn_fc_w,
           weight_neighbor, k):
    return _run(prev_h, emb_rel, edge_index, rid, pos_proj_w, attn_fc_w,
                weight_neighbor)


# speedup vs baseline: 1.4146x; 1.0036x over previous
import jax
import jax.numpy as jnp
from jax import lax
from jax.experimental import pallas as pl
from jax.experimental.pallas import tpu as pltpu
from jax.experimental.pallas import tpu_sc as plsc

N = 10000
E = 160000
D = 256
H = 128
NR = 200

NC = 2
NS = 16
NW = NC * NS

ROWB = 1000

CA = 96
NKA = 53
EPADA = NW * NKA * CA

CB = 80
NKB = 125
NKBP = 128
GB = 4
NGB = NKBP // GB


def _dense_body(x_ref, w_ref, p1_ref, p2_ref, m1a_ref, m1b_ref):
    z = jnp.dot(x_ref[...], w_ref[...], preferred_element_type=jnp.float32)
    p1_ref[...] = z[:, :D]
    p2_ref[...] = z[:, D:2 * D]
    m1a_ref[...] = z[:, 2 * D:2 * D + H]
    m1b_ref[...] = z[:, 2 * D + H:]


def _dense_tables(prev_h, w_all):
    return pl.pallas_call(
        _dense_body,
        grid=(N // ROWB,),
        in_specs=[
            pl.BlockSpec((ROWB, D), lambda i: (i, 0)),
            pl.BlockSpec((D, 3 * D), lambda i: (0, 0)),
        ],
        out_specs=[
            pl.BlockSpec((ROWB, D), lambda i: (i, 0)),
            pl.BlockSpec((ROWB, D), lambda i: (i, 0)),
            pl.BlockSpec((ROWB, H), lambda i: (i, 0)),
            pl.BlockSpec((ROWB, H), lambda i: (i, 0)),
        ],
        out_shape=[
            jax.ShapeDtypeStruct((N, D), jnp.float32),
            jax.ShapeDtypeStruct((N, D), jnp.float32),
            jax.ShapeDtypeStruct((N, H), jnp.float32),
            jax.ShapeDtypeStruct((N, H), jnp.float32),
        ],
    )(prev_h, w_all)


def _rel_body(x_ref, w_ref, ra_ref, rb_ref):
    z = jnp.dot(x_ref[...], w_ref[...], preferred_element_type=jnp.float32)
    ra_ref[...] = z[:, :H]
    rb_ref[...] = z[:, H:]


def _rel_tables(emb_rel, wn2):
    return pl.pallas_call(
        _rel_body,
        out_shape=[
            jax.ShapeDtypeStruct((NR, H), jnp.float32),
            jax.ShapeDtypeStruct((NR, H), jnp.float32),
        ],
    )(emb_rel, wn2)


def _attn_body(pa_hbm, p1_hbm, p2_hbm, wv_hbm, ex_hbm,
               pab, p1b0, p2b0, p1b1, p2b1, eb, wv, hbuf, sem0, sem1):
    c = lax.axis_index("c")
    s = lax.axis_index("s")
    wid = s * NC + c
    pltpu.sync_copy(wv_hbm, wv)
    pltpu.sync_copy(pa_hbm.at[pl.ds(wid * (2 * NKA * CA), 2 * NKA * CA)], pab)
    bufs = ((p1b0, p2b0, sem0), (p1b1, p2b1, sem1))
    lanes = lax.iota(jnp.int32, 16)
    for i in range(16):
        hbuf[pl.ds(32 * i + 16, 16)] = jnp.zeros((16,), jnp.float32)

    def issue(b, k):
        p1b, p2b, sem = bufs[b]

        @pl.when(k < NKA)
        def _():
            pltpu.async_copy(
                p1_hbm.at[pab.at[pl.ds(2 * CA * k, CA)]], p1b, sem)
            pltpu.async_copy(
                p2_hbm.at[pab.at[pl.ds(2 * CA * k + CA, CA)]], p2b, sem)

    def compute(b, k):
        p1b, p2b, sem = bufs[b]

        @pl.when(k < NKA)
        def _():
            pltpu.make_async_copy(
                p1_hbm.at[pab.at[pl.ds(0, CA)]], p1b, sem).wait()
            pltpu.make_async_copy(
                p2_hbm.at[pab.at[pl.ds(0, CA)]], p2b, sem).wait()

            def grp(g, carry):
                v = jnp.zeros((16,), jnp.float32)
                for i in range(16):
                    r = g * 16 + i
                    acc = jnp.zeros((16,), jnp.float32)
                    for j in range(D // 16):
                        a = (p1b[r, pl.ds(16 * j, 16)]
                             + p2b[r, pl.ds(16 * j, 16)])
                        a = jnp.maximum(a, 0.01 * a)
                        acc = acc + a * wv[pl.ds(16 * j, 16)]
                    slot = 32 * i
                    s = acc
                    for sh in (8, 4, 2, 1):
                        hbuf[pl.ds(slot, 16)] = s
                        s = s + hbuf[pl.ds(slot + sh, 16)]
                    v = jnp.where(lanes == i, s[0], v)
                eb[pl.ds(16 * g, 16)] = jnp.exp(v)
                return carry

            lax.fori_loop(0, CA // 16, grp, 0)
            pltpu.sync_copy(eb, ex_hbm.at[pl.ds((wid * NKA + k) * CA, CA)])

    issue(0, 0)

    def pair(p, carry):
        k0 = 2 * p
        issue(1, k0 + 1)
        compute(0, k0)
        issue(0, k0 + 2)
        compute(1, k0 + 1)
        return carry

    lax.fori_loop(0, (NKA + 1) // 2, pair, 0)


def _attn_pass(pa, p1, p2, wv):
    mesh = plsc.VectorSubcoreMesh(core_axis_name="c", subcore_axis_name="s")
    f = pl.kernel(
        _attn_body,
        out_type=jax.ShapeDtypeStruct((EPADA,), jnp.float32),
        mesh=mesh,
        scratch_types=[
            pltpu.VMEM((2 * NKA * CA,), jnp.int32),
            pltpu.VMEM((CA, D), jnp.float32),
            pltpu.VMEM((CA, D), jnp.float32),
            pltpu.VMEM((CA, D), jnp.float32),
            pltpu.VMEM((CA, D), jnp.float32),
            pltpu.VMEM((CA,), jnp.float32),
            pltpu.VMEM((D,), jnp.float32),
            pltpu.VMEM((512,), jnp.float32),
            pltpu.SemaphoreType.DMA,
            pltpu.SemaphoreType.DMA,
        ],
    )
    return f(pa, p1, p2, wv)


ZR = 25
RPS = N // NS
DPAD = 10240


def _agg_body(pb_hbm, pe_hbm, m1a_hbm, m1b_hbm, r2a_hbm, r2b_hbm,
              outa_hbm, outb_hbm,
              gb0, ge0, gb1, ge1, m1v0, relb0, m1v1, relb1, dstw,
              zb, db, dvb, acc_sh, den_sh,
              isem0, isem1, sem0, sem1):
    c = lax.axis_index("c")
    s = lax.axis_index("s")

    def zrow_body(i, carry):
        for j in range(H // 16):
            zb[i, pl.ds(16 * j, 16)] = jnp.zeros((16,), jnp.float32)
        return carry

    lax.fori_loop(0, 16, zrow_body, 0)

    def dz_body(i, carry):
        db[pl.ds(16 * i, 16)] = jnp.zeros((16,), jnp.float32)
        return carry

    lax.fori_loop(0, 8, dz_body, 0)

    def zcp_body(t, carry):
        bid = s + t * NS

        @pl.when(bid < N // 16)
        def _():
            pltpu.sync_copy(zb, acc_sh.at[pl.ds(bid * 16, 16)])
        return carry

    lax.fori_loop(0, (N // 16 + NS - 1) // NS, zcp_body, 0)

    def dcp_body(kk, carry):
        pltpu.sync_copy(db, den_sh.at[pl.ds(s * (DPAD // NS) + kk * 128, 128)])
        return carry

    lax.fori_loop(0, (DPAD // NS) // 128, dcp_body, 0)
    plsc.subcore_barrier()

    gbufs = ((gb0, ge0, isem0), (gb1, ge1, isem1))
    cbufs = ((m1v0, relb0, sem0), (m1v1, relb1, sem1))

    def fetchg(gbi, gi):
        gb, ge, isem = gbufs[gbi]

        @pl.when(gi < NGB)
        def _():
            pltpu.async_copy(
                pb_hbm.at[pl.ds((s * NKBP + GB * gi) * 3 * CB, 3 * GB * CB)],
                gb, isem)
            pltpu.async_copy(
                pe_hbm.at[pl.ds((s * NKBP + GB * gi) * CB, GB * CB)],
                ge, isem)

    def waitg(gbi):
        gb, ge, isem = gbufs[gbi]
        pltpu.make_async_copy(
            pb_hbm.at[pl.ds(0, 3 * GB * CB)], gb, isem).wait()
        pltpu.make_async_copy(
            pe_hbm.at[pl.ds(0, GB * CB)], ge, isem).wait()

    def issue(cbi, gbi, jj, k):
        m1v, relb, sem = cbufs[cbi]
        gb, _, _ = gbufs[gbi]

        @pl.when(k < NKB)
        def _():
            srcix = gb.at[pl.ds(3 * CB * jj, CB)]
            ridix = gb.at[pl.ds(3 * CB * jj + 2 * CB, CB)]

            @pl.when(c == 0)
            def _():
                pltpu.async_copy(m1a_hbm.at[srcix], m1v, sem)
                pltpu.async_copy(r2a_hbm.at[ridix], relb, sem)

            @pl.when(c == 1)
            def _():
                pltpu.async_copy(m1b_hbm.at[srcix], m1v, sem)
                pltpu.async_copy(r2b_hbm.at[ridix], relb, sem)

    def compute(cbi, gbi, jj, k):
        m1v, relb, sem = cbufs[cbi]
        gb, ge, _ = gbufs[gbi]

        @pl.when(k < NKB)
        def _():
            dummy = gb.at[pl.ds(0, CB)]
            pltpu.make_async_copy(m1a_hbm.at[dummy], m1v, sem).wait()
            pltpu.make_async_copy(r2a_hbm.at[dummy], relb, sem).wait()

            def grp_body(g2, carry2):
                off = 3 * CB * jj + 16 * g2
                exv = ge[pl.ds(CB * jj + 16 * g2, 16)]
                dstw[pl.ds(16 * g2, 16)] = gb[pl.ds(off + CB, 16)]
                base = 16 * g2
                for i in range(16):
                    exi = jnp.full((16,), exv[i], jnp.float32)
                    r = base + i
                    for j in range(H // 16):
                        m1v[r, pl.ds(16 * j, 16)] = (
                            m1v[r, pl.ds(16 * j, 16)]
                            + relb[r, pl.ds(16 * j, 16)]
                        ) * exi
                return carry2

            lax.fori_loop(0, CB // 16, grp_body, 0)
            pltpu.sync_copy(m1v, acc_sh.at[dstw], add=True)
            pltpu.sync_copy(ge.at[pl.ds(CB * jj, CB)], den_sh.at[dstw],
                            add=True)

    fetchg(0, 0)
    fetchg(1, 1)

    def pairg(p, carry):
        g0 = 2 * p
        g1 = 2 * p + 1
        waitg(0)
        issue(0, 0, 0, g0 * GB + 0)
        issue(1, 0, 1, g0 * GB + 1)
        compute(0, 0, 0, g0 * GB + 0)
        issue(0, 0, 2, g0 * GB + 2)
        compute(1, 0, 1, g0 * GB + 1)
        issue(1, 0, 3, g0 * GB + 3)
        compute(0, 0, 2, g0 * GB + 2)
        waitg(1)
        issue(0, 1, 0, g1 * GB + 0)
        compute(1, 0, 3, g0 * GB + 3)
        fetchg(0, g0 + 2)
        issue(1, 1, 1, g1 * GB + 1)
        compute(0, 1, 0, g1 * GB + 0)
        issue(0, 1, 2, g1 * GB + 2)
        compute(1, 1, 1, g1 * GB + 1)
        issue(1, 1, 3, g1 * GB + 3)
        compute(0, 1, 2, g1 * GB + 2)
        compute(1, 1, 3, g1 * GB + 3)
        fetchg(1, g1 + 2)
        return carry

    lax.fori_loop(0, NGB // 2, pairg, 0)
    plsc.subcore_barrier()

    nblk = N // 16

    def wblk_body(t, carry):
        bid = s + t * NS

        @pl.when(bid < nblk)
        def _():
            r0 = bid * 16
            pltpu.sync_copy(acc_sh.at[pl.ds(r0, 16)], zb)
            pltpu.sync_copy(den_sh.at[pl.ds(r0, 16)], dvb)
            dv = dvb[pl.ds(0, 16)]
            rv16 = jnp.where(dv == 0.0, 0.0, 1.0 / dv)
            for i in range(16):
                rv = jnp.full((16,), rv16[i], jnp.float32)
                for j in range(H // 16):
                    zb[i, pl.ds(16 * j, 16)] = zb[i, pl.ds(16 * j, 16)] * rv

            @pl.when(c == 0)
            def _():
                pltpu.sync_copy(zb, outa_hbm.at[pl.ds(r0, 16)])

            @pl.when(c == 1)
            def _():
                pltpu.sync_copy(zb, outb_hbm.at[pl.ds(r0, 16)])

        return carry

    lax.fori_loop(0, (nblk + NS - 1) // NS, wblk_body, 0)


def _agg_pass(pb, pe, m1a, m1b, r2a, r2b):
    mesh = plsc.VectorSubcoreMesh(core_axis_name="c", subcore_axis_name="s")
    f = pl.kernel(
        _agg_body,
        out_type=[
            jax.ShapeDtypeStruct((N, H), jnp.float32),
            jax.ShapeDtypeStruct((N, H), jnp.float32),
        ],
        mesh=mesh,
        scratch_types=[
            pltpu.VMEM((3 * GB * CB,), jnp.int32),
            pltpu.VMEM((GB * CB,), jnp.float32),
            pltpu.VMEM((3 * GB * CB,), jnp.int32),
            pltpu.VMEM((GB * CB,), jnp.float32),
            pltpu.VMEM((CB, H), jnp.float32),
            pltpu.VMEM((CB, H), jnp.float32),
            pltpu.VMEM((CB, H), jnp.float32),
            pltpu.VMEM((CB, H), jnp.float32),
            pltpu.VMEM((CB,), jnp.int32),
            pltpu.VMEM((16, H), jnp.float32),
            pltpu.VMEM((128,), jnp.float32),
            pltpu.VMEM((16,), jnp.float32),
            pltpu.VMEM_SHARED((N, H), jnp.float32),
            pltpu.VMEM_SHARED((DPAD,), jnp.float32),
            pltpu.SemaphoreType.DMA,
            pltpu.SemaphoreType.DMA,
            pltpu.SemaphoreType.DMA,
            pltpu.SemaphoreType.DMA,
        ],
    )
    return f(pb, pe, m1a, m1b, r2a, r2b)


@jax.jit
def _run(prev_h, emb_rel, edge_index, rid, pos_proj_w, attn_fc_w,
         weight_neighbor):
    w_all = jnp.concatenate(
        [pos_proj_w[:D], pos_proj_w[D:], weight_neighbor[:D]], axis=1)
    p1, p2, m1a, m1b = _dense_tables(prev_h, w_all)
    r2a, r2b = _rel_tables(emb_rel, weight_neighbor[D:])
    src = edge_index[0]
    dst = edge_index[1]

    def pad_chunks_a(x):
        xw = x.reshape(NW, E // NW)
        xw = jnp.pad(xw, ((0, 0), (0, NKA * CA - E // NW)))
        return xw.reshape(NW, NKA, 1, CA)

    pa = jnp.concatenate([pad_chunks_a(src), pad_chunks_a(dst)], axis=2)
    pa = pa.reshape(NW * NKA * 2 * CA)
    ex = _attn_pass(pa, p1, p2, attn_fc_w[:, 0])
    ex = ex.reshape(NW, NKA * CA)[:, :E // NW].reshape(E)

    def pad_chunks_b(x):
        xw = x.reshape(NS, NKB, 1, CB)
        return jnp.pad(xw, ((0, 0), (0, NKBP - NKB), (0, 0), (0, 0)))

    pb = jnp.concatenate(
        [pad_chunks_b(src), pad_chunks_b(dst), pad_chunks_b(rid)], axis=2)
    pb = pb.reshape(NS * NKBP * 3 * CB)
    pe = pad_chunks_b(ex).reshape(NS * NKBP * CB)
    outa, outb = _agg_pass(pb, pe, m1a, m1b, r2a, r2b)
    return jnp.concatenate([outa, outb], axis=1)


def kernel(prev_h, emb_rel, edge_index, rid, pos_proj_w, attn_fc_w,
           weight_neighbor, k):
    return _run(prev_h, emb_rel, edge_index, rid, pos_proj_w, attn_fc_w,
                weight_neighbor)


# speedup vs baseline: 1.5434x; 1.0911x over previous
import jax
import jax.numpy as jnp
from jax import lax
from jax.experimental import pallas as pl
from jax.experimental.pallas import tpu as pltpu
from jax.experimental.pallas import tpu_sc as plsc

N = 10000
E = 160000
D = 256
H = 128
NR = 200

NC = 2
NS = 16
NW = NC * NS

ROWB = 1000

CA = 64
NKA = 79
EPADA = NW * NKA * CA

CB = 80
NKB = 125
NKBP = 128
GB = 4
NGB = NKBP // GB


def _dense_body(x_ref, w_ref, p1_ref, p2_ref, m1a_ref, m1b_ref):
    z = jnp.dot(x_ref[...], w_ref[...], preferred_element_type=jnp.float32)
    p1_ref[...] = z[:, :D]
    p2_ref[...] = z[:, D:2 * D]
    m1a_ref[...] = z[:, 2 * D:2 * D + H]
    m1b_ref[...] = z[:, 2 * D + H:]


def _dense_tables(prev_h, w_all):
    return pl.pallas_call(
        _dense_body,
        grid=(N // ROWB,),
        in_specs=[
            pl.BlockSpec((ROWB, D), lambda i: (i, 0)),
            pl.BlockSpec((D, 3 * D), lambda i: (0, 0)),
        ],
        out_specs=[
            pl.BlockSpec((ROWB, D), lambda i: (i, 0)),
            pl.BlockSpec((ROWB, D), lambda i: (i, 0)),
            pl.BlockSpec((ROWB, H), lambda i: (i, 0)),
            pl.BlockSpec((ROWB, H), lambda i: (i, 0)),
        ],
        out_shape=[
            jax.ShapeDtypeStruct((N, D), jnp.float32),
            jax.ShapeDtypeStruct((N, D), jnp.float32),
            jax.ShapeDtypeStruct((N, H), jnp.float32),
            jax.ShapeDtypeStruct((N, H), jnp.float32),
        ],
    )(prev_h, w_all)


def _rel_body(x_ref, w_ref, ra_ref, rb_ref):
    z = jnp.dot(x_ref[...], w_ref[...], preferred_element_type=jnp.float32)
    ra_ref[...] = z[:, :H]
    rb_ref[...] = z[:, H:]


def _rel_tables(emb_rel, wn2):
    return pl.pallas_call(
        _rel_body,
        out_shape=[
            jax.ShapeDtypeStruct((NR, H), jnp.float32),
            jax.ShapeDtypeStruct((NR, H), jnp.float32),
        ],
    )(emb_rel, wn2)


def _attn_body(pa_hbm, p1_hbm, p2_hbm, wv_hbm, ex_hbm,
               pab, p1b0, p2b0, eb0, p1b1, p2b1, eb1, p1b2, p2b2, eb2,
               wv, hbuf,
               sem0, sem1, sem2, osem0, osem1, osem2):
    c = lax.axis_index("c")
    s = lax.axis_index("s")
    wid = s * NC + c
    pltpu.sync_copy(wv_hbm, wv)
    pltpu.sync_copy(pa_hbm.at[pl.ds(wid * (2 * NKA * CA), 2 * NKA * CA)], pab)
    bufs = ((p1b0, p2b0, eb0, sem0, osem0),
            (p1b1, p2b1, eb1, sem1, osem1),
            (p1b2, p2b2, eb2, sem2, osem2))
    lanes = lax.iota(jnp.int32, 16)
    for i in range(16):
        hbuf[pl.ds(32 * i + 16, 16)] = jnp.zeros((16,), jnp.float32)

    def issue(b, k):
        p1b, p2b, _, sem, _ = bufs[b]

        @pl.when(k < NKA)
        def _():
            pltpu.async_copy(
                p1_hbm.at[pab.at[pl.ds(2 * CA * k, CA)]], p1b, sem)
            pltpu.async_copy(
                p2_hbm.at[pab.at[pl.ds(2 * CA * k + CA, CA)]], p2b, sem)

    def compute(b, k):
        p1b, p2b, eb, sem, osem = bufs[b]

        @pl.when(k < NKA)
        def _():
            pltpu.make_async_copy(
                p1_hbm.at[pab.at[pl.ds(0, CA)]], p1b, sem).wait()
            pltpu.make_async_copy(
                p2_hbm.at[pab.at[pl.ds(0, CA)]], p2b, sem).wait()

            @pl.when(k >= 3)
            def _():
                pltpu.make_async_copy(
                    eb, ex_hbm.at[pl.ds(0, CA)], osem).wait()

            def grp(g, carry):
                v = jnp.zeros((16,), jnp.float32)
                for i in range(16):
                    r = g * 16 + i
                    acc = jnp.zeros((16,), jnp.float32)
                    for j in range(D // 16):
                        a = (p1b[r, pl.ds(16 * j, 16)]
                             + p2b[r, pl.ds(16 * j, 16)])
                        a = jnp.maximum(a, 0.01 * a)
                        acc = acc + a * wv[pl.ds(16 * j, 16)]
                    slot = 32 * i
                    s2 = acc
                    for sh in (8, 4, 2, 1):
                        hbuf[pl.ds(slot, 16)] = s2
                        s2 = s2 + hbuf[pl.ds(slot + sh, 16)]
                    v = jnp.where(lanes == i, s2[0], v)
                eb[pl.ds(16 * g, 16)] = jnp.exp(v)
                return carry

            lax.fori_loop(0, CA // 16, grp, 0)
            pltpu.async_copy(
                eb, ex_hbm.at[pl.ds((wid * NKA + k) * CA, CA)], osem)

    issue(0, 0)
    issue(1, 1)
    issue(2, 2)

    def trip(p, carry):
        k0 = 3 * p
        compute(0, k0)
        issue(0, k0 + 3)
        compute(1, k0 + 1)
        issue(1, k0 + 4)
        compute(2, k0 + 2)
        issue(2, k0 + 5)
        return carry

    lax.fori_loop(0, (NKA + 2) // 3, trip, 0)
    for b in range(3):
        _, _, eb, _, osem = bufs[b]
        pltpu.make_async_copy(eb, ex_hbm.at[pl.ds(0, CA)], osem).wait()


def _attn_pass(pa, p1, p2, wv):
    mesh = plsc.VectorSubcoreMesh(core_axis_name="c", subcore_axis_name="s")
    f = pl.kernel(
        _attn_body,
        out_type=jax.ShapeDtypeStruct((EPADA,), jnp.float32),
        mesh=mesh,
        scratch_types=[
            pltpu.VMEM((2 * NKA * CA,), jnp.int32),
            pltpu.VMEM((CA, D), jnp.float32),
            pltpu.VMEM((CA, D), jnp.float32),
            pltpu.VMEM((CA,), jnp.float32),
            pltpu.VMEM((CA, D), jnp.float32),
            pltpu.VMEM((CA, D), jnp.float32),
            pltpu.VMEM((CA,), jnp.float32),
            pltpu.VMEM((CA, D), jnp.float32),
            pltpu.VMEM((CA, D), jnp.float32),
            pltpu.VMEM((CA,), jnp.float32),
            pltpu.VMEM((D,), jnp.float32),
            pltpu.VMEM((512,), jnp.float32),
            pltpu.SemaphoreType.DMA,
            pltpu.SemaphoreType.DMA,
            pltpu.SemaphoreType.DMA,
            pltpu.SemaphoreType.DMA,
            pltpu.SemaphoreType.DMA,
            pltpu.SemaphoreType.DMA,
        ],
    )
    return f(pa, p1, p2, wv)


ZR = 25
RPS = N // NS
DPAD = 10240


def _agg_body(pb_hbm, pe_hbm, m1a_hbm, m1b_hbm, r2a_hbm, r2b_hbm,
              outa_hbm, outb_hbm,
              gb0, ge0, gb1, ge1, m1v0, relb0, m1v1, relb1, dstw,
              zb, db, dvb, acc_sh, den_sh,
              isem0, isem1, sem0, sem1):
    c = lax.axis_index("c")
    s = lax.axis_index("s")

    def zrow_body(i, carry):
        for j in range(H // 16):
            zb[i, pl.ds(16 * j, 16)] = jnp.zeros((16,), jnp.float32)
        return carry

    lax.fori_loop(0, 16, zrow_body, 0)

    def dz_body(i, carry):
        db[pl.ds(16 * i, 16)] = jnp.zeros((16,), jnp.float32)
        return carry

    lax.fori_loop(0, 8, dz_body, 0)

    def zcp_body(t, carry):
        bid = s + t * NS

        @pl.when(bid < N // 16)
        def _():
            pltpu.sync_copy(zb, acc_sh.at[pl.ds(bid * 16, 16)])
        return carry

    lax.fori_loop(0, (N // 16 + NS - 1) // NS, zcp_body, 0)

    def dcp_body(kk, carry):
        pltpu.sync_copy(db, den_sh.at[pl.ds(s * (DPAD // NS) + kk * 128, 128)])
        return carry

    lax.fori_loop(0, (DPAD // NS) // 128, dcp_body, 0)
    plsc.subcore_barrier()

    gbufs = ((gb0, ge0, isem0), (gb1, ge1, isem1))
    cbufs = ((m1v0, relb0, sem0), (m1v1, relb1, sem1))

    def fetchg(gbi, gi):
        gb, ge, isem = gbufs[gbi]

        @pl.when(gi < NGB)
        def _():
            pltpu.async_copy(
                pb_hbm.at[pl.ds((s * NKBP + GB * gi) * 3 * CB, 3 * GB * CB)],
                gb, isem)
            pltpu.async_copy(
                pe_hbm.at[pl.ds((s * NKBP + GB * gi) * CB, GB * CB)],
                ge, isem)

    def waitg(gbi):
        gb, ge, isem = gbufs[gbi]
        pltpu.make_async_copy(
            pb_hbm.at[pl.ds(0, 3 * GB * CB)], gb, isem).wait()
        pltpu.make_async_copy(
            pe_hbm.at[pl.ds(0, GB * CB)], ge, isem).wait()

    def issue(cbi, gbi, jj, k):
        m1v, relb, sem = cbufs[cbi]
        gb, _, _ = gbufs[gbi]

        @pl.when(k < NKB)
        def _():
            srcix = gb.at[pl.ds(3 * CB * jj, CB)]
            ridix = gb.at[pl.ds(3 * CB * jj + 2 * CB, CB)]

            @pl.when(c == 0)
            def _():
                pltpu.async_copy(m1a_hbm.at[srcix], m1v, sem)
                pltpu.async_copy(r2a_hbm.at[ridix], relb, sem)

            @pl.when(c == 1)
            def _():
                pltpu.async_copy(m1b_hbm.at[srcix], m1v, sem)
                pltpu.async_copy(r2b_hbm.at[ridix], relb, sem)

    def compute(cbi, gbi, jj, k):
        m1v, relb, sem = cbufs[cbi]
        gb, ge, _ = gbufs[gbi]

        @pl.when(k < NKB)
        def _():
            dummy = gb.at[pl.ds(0, CB)]
            pltpu.make_async_copy(m1a_hbm.at[dummy], m1v, sem).wait()
            pltpu.make_async_copy(r2a_hbm.at[dummy], relb, sem).wait()

            def grp_body(g2, carry2):
                off = 3 * CB * jj + 16 * g2
                exv = ge[pl.ds(CB * jj + 16 * g2, 16)]
                dstw[pl.ds(16 * g2, 16)] = gb[pl.ds(off + CB, 16)]
                base = 16 * g2
                for i in range(16):
                    exi = jnp.full((16,), exv[i], jnp.float32)
                    r = base + i
                    for j in range(H // 16):
                        m1v[r, pl.ds(16 * j, 16)] = (
                            m1v[r, pl.ds(16 * j, 16)]
                            + relb[r, pl.ds(16 * j, 16)]
                        ) * exi
                return carry2

            lax.fori_loop(0, CB // 16, grp_body, 0)
            pltpu.sync_copy(m1v, acc_sh.at[dstw], add=True)
            pltpu.sync_copy(ge.at[pl.ds(CB * jj, CB)], den_sh.at[dstw],
                            add=True)

    fetchg(0, 0)
    fetchg(1, 1)

    def pairg(p, carry):
        g0 = 2 * p
        g1 = 2 * p + 1
        waitg(0)
        issue(0, 0, 0, g0 * GB + 0)
        issue(1, 0, 1, g0 * GB + 1)
        compute(0, 0, 0, g0 * GB + 0)
        issue(0, 0, 2, g0 * GB + 2)
        compute(1, 0, 1, g0 * GB + 1)
        issue(1, 0, 3, g0 * GB + 3)
        compute(0, 0, 2, g0 * GB + 2)
        waitg(1)
        issue(0, 1, 0, g1 * GB + 0)
        compute(1, 0, 3, g0 * GB + 3)
        fetchg(0, g0 + 2)
        issue(1, 1, 1, g1 * GB + 1)
        compute(0, 1, 0, g1 * GB + 0)
        issue(0, 1, 2, g1 * GB + 2)
        compute(1, 1, 1, g1 * GB + 1)
        issue(1, 1, 3, g1 * GB + 3)
        compute(0, 1, 2, g1 * GB + 2)
        compute(1, 1, 3, g1 * GB + 3)
        fetchg(1, g1 + 2)
        return carry

    lax.fori_loop(0, NGB // 2, pairg, 0)
    plsc.subcore_barrier()

    nblk = N // 16

    def wblk_body(t, carry):
        bid = s + t * NS

        @pl.when(bid < nblk)
        def _():
            r0 = bid * 16
            pltpu.sync_copy(acc_sh.at[pl.ds(r0, 16)], zb)
            pltpu.sync_copy(den_sh.at[pl.ds(r0, 16)], dvb)
            dv = dvb[pl.ds(0, 16)]
            rv16 = jnp.where(dv == 0.0, 0.0, 1.0 / dv)
            for i in range(16):
                rv = jnp.full((16,), rv16[i], jnp.float32)
                for j in range(H // 16):
                    zb[i, pl.ds(16 * j, 16)] = zb[i, pl.ds(16 * j, 16)] * rv

            @pl.when(c == 0)
            def _():
                pltpu.sync_copy(zb, outa_hbm.at[pl.ds(r0, 16)])

            @pl.when(c == 1)
            def _():
                pltpu.sync_copy(zb, outb_hbm.at[pl.ds(r0, 16)])

        return carry

    lax.fori_loop(0, (nblk + NS - 1) // NS, wblk_body, 0)


def _agg_pass(pb, pe, m1a, m1b, r2a, r2b):
    mesh = plsc.VectorSubcoreMesh(core_axis_name="c", subcore_axis_name="s")
    f = pl.kernel(
        _agg_body,
        out_type=[
            jax.ShapeDtypeStruct((N, H), jnp.float32),
            jax.ShapeDtypeStruct((N, H), jnp.float32),
        ],
        mesh=mesh,
        scratch_types=[
            pltpu.VMEM((3 * GB * CB,), jnp.int32),
            pltpu.VMEM((GB * CB,), jnp.float32),
            pltpu.VMEM((3 * GB * CB,), jnp.int32),
            pltpu.VMEM((GB * CB,), jnp.float32),
            pltpu.VMEM((CB, H), jnp.float32),
            pltpu.VMEM((CB, H), jnp.float32),
            pltpu.VMEM((CB, H), jnp.float32),
            pltpu.VMEM((CB, H), jnp.float32),
            pltpu.VMEM((CB,), jnp.int32),
            pltpu.VMEM((16, H), jnp.float32),
            pltpu.VMEM((128,), jnp.float32),
            pltpu.VMEM((16,), jnp.float32),
            pltpu.VMEM_SHARED((N, H), jnp.float32),
            pltpu.VMEM_SHARED((DPAD,), jnp.float32),
            pltpu.SemaphoreType.DMA,
            pltpu.SemaphoreType.DMA,
            pltpu.SemaphoreType.DMA,
            pltpu.SemaphoreType.DMA,
        ],
    )
    return f(pb, pe, m1a, m1b, r2a, r2b)


@jax.jit
def _run(prev_h, emb_rel, edge_index, rid, pos_proj_w, attn_fc_w,
         weight_neighbor):
    w_all = jnp.concatenate(
        [pos_proj_w[:D], pos_proj_w[D:], weight_neighbor[:D]], axis=1)
    p1, p2, m1a, m1b = _dense_tables(prev_h, w_all)
    r2a, r2b = _rel_tables(emb_rel, weight_neighbor[D:])
    src = edge_index[0]
    dst = edge_index[1]

    def pad_chunks_a(x):
        xw = x.reshape(NW, E // NW)
        xw = jnp.pad(xw, ((0, 0), (0, NKA * CA - E // NW)))
        return xw.reshape(NW, NKA, 1, CA)

    pa = jnp.concatenate([pad_chunks_a(src), pad_chunks_a(dst)], axis=2)
    pa = pa.reshape(NW * NKA * 2 * CA)
    ex = _attn_pass(pa, p1, p2, attn_fc_w[:, 0])
    ex = ex.reshape(NW, NKA * CA)[:, :E // NW].reshape(E)

    def pad_chunks_b(x):
        xw = x.reshape(NS, NKB, 1, CB)
        return jnp.pad(xw, ((0, 0), (0, NKBP - NKB), (0, 0), (0, 0)))

    pb = jnp.concatenate(
        [pad_chunks_b(src), pad_chunks_b(dst), pad_chunks_b(rid)], axis=2)
    pb = pb.reshape(NS * NKBP * 3 * CB)
    pe = pad_chunks_b(ex).reshape(NS * NKBP * CB)
    outa, outb = _agg_pass(pb, pe, m1a, m1b, r2a, r2b)
    return jnp.concatenate([outa, outb], axis=1)


def kernel(prev_h, emb_rel, edge_index, rid, pos_proj_w, attn_fc_w,
           weight_neighbor, k):
    return _run(prev_h, emb_rel, edge_index, rid, pos_proj_w, attn_fc_w,
                weight_neighbor)
